# Initial kernel scaffold; baseline (speedup 1.0000x reference)
import jax
import jax.numpy as jnp
from jax.experimental import pallas as pl


def kernel(x, batch):
    raise NotImplementedError("write your pallas kernel here")


# speedup vs baseline: 23.8150x; 23.8150x over previous
import functools

import jax
import jax.numpy as jnp
from jax import lax
from jax.experimental import pallas as pl
from jax.experimental.pallas import tpu as pltpu
from jax.experimental.pallas import tpu_sc as plsc

N = 6400000
NUM_SEG = 512
NC, NS, L = 2, 16, 16
NW = NC * NS
B = 12800
NBLK = N // B
STEPS = NBLK // NW
EXTRA = NBLK % NW
SEG_PER_TILE = NUM_SEG // NS


def _make_kernel():
    mesh = plsc.VectorSubcoreMesh(core_axis_name="c", subcore_axis_name="s")

    @functools.partial(
        pl.kernel,
        mesh=mesh,
        compiler_params=pltpu.CompilerParams(needs_layout_passes=False),
        out_type=jax.ShapeDtypeStruct((NC * NUM_SEG,), jnp.float32),
        scratch_types=[
            pltpu.VMEM((2, B), jnp.float32),
            pltpu.VMEM((2, B), jnp.int32),
            pltpu.VMEM((NUM_SEG,), jnp.float32),
            pltpu.VMEM((NS * SEG_PER_TILE,), jnp.float32),
            pltpu.VMEM((SEG_PER_TILE,), jnp.float32),
            pltpu.VMEM_SHARED((NS * NUM_SEG,), jnp.float32),
            pltpu.SemaphoreType.DMA,
            pltpu.SemaphoreType.DMA,
            pltpu.SemaphoreType.DMA,
            pltpu.SemaphoreType.DMA,
        ],
    )
    def seg_sum(x_hbm, b_hbm, out_hbm, xbuf, bbuf, acc, tmp, res, shared,
                sx0, sb0, sx1, sb1):
        cid = lax.axis_index("c")
        sid = lax.axis_index("s")
        wid = sid * NC + cid

        def zbody(i, _):
            acc[pl.ds(i * L, L)] = jnp.zeros((L,), jnp.float32)
            return _
        lax.fori_loop(0, NUM_SEG // L, zbody, 0)

        sems = ((sx0, sb0), (sx1, sb1))

        def dma_start(step, buf):
            off = (wid + NW * step) * B
            pltpu.async_copy(x_hbm.at[pl.ds(off, B)], xbuf.at[buf],
                             sems[buf][0])
            pltpu.async_copy(b_hbm.at[pl.ds(off, B)], bbuf.at[buf],
                             sems[buf][1])

        def dma_wait(buf):
            pltpu.make_async_copy(x_hbm.at[pl.ds(0, B)], xbuf.at[buf],
                                  sems[buf][0]).wait()
            pltpu.make_async_copy(b_hbm.at[pl.ds(0, B)], bbuf.at[buf],
                                  sems[buf][1]).wait()

        def compute(buf):
            def vbody(i, _):
                off = i * L
                bv = bbuf[buf, pl.ds(off, L)]
                xv = xbuf[buf, pl.ds(off, L)]
                plsc.addupdate_scatter(acc, [bv], xv)
                return _
            lax.fori_loop(0, B // L, vbody, 0)

        total_steps = STEPS + 1
        dma_start(0, 0)
        for step in range(total_steps):
            buf = step & 1
            if step == total_steps - 1:
                @pl.when(wid < EXTRA)
                def _():
                    dma_wait(buf)
                    compute(buf)
            else:
                dma_wait(buf)
                nxt = step + 1
                if nxt == total_steps - 1:
                    @pl.when(wid < EXTRA)
                    def _():
                        dma_start(nxt, 1 - buf)
                else:
                    dma_start(nxt, 1 - buf)
                compute(buf)

        pltpu.sync_copy(acc, shared.at[pl.ds(sid * NUM_SEG, NUM_SEG)])
        plsc.subcore_barrier()
        for w in range(NS):
            pltpu.sync_copy(
                shared.at[pl.ds(w * NUM_SEG + sid * SEG_PER_TILE,
                                SEG_PER_TILE)],
                tmp.at[pl.ds(w * SEG_PER_TILE, SEG_PER_TILE)])
        for half in range(SEG_PER_TILE // L):
            v = jnp.zeros((L,), jnp.float32)
            for w in range(NS):
                v = v + tmp[pl.ds(w * SEG_PER_TILE + half * L, L)]
            res[pl.ds(half * L, L)] = v
        pltpu.sync_copy(res, out_hbm.at[pl.ds(cid * NUM_SEG + sid * SEG_PER_TILE,
                                              SEG_PER_TILE)])

    return seg_sum


_seg_sum = _make_kernel()


def kernel(x, batch):
    partials = _seg_sum(x.reshape(N), batch.astype(jnp.int32)).reshape(NC, NUM_SEG)
    return (partials[0] + partials[1]).reshape(NUM_SEG, 1)


# speedup vs baseline: 53.2195x; 2.2347x over previous
import functools

import jax
import jax.numpy as jnp
from jax import lax
from jax.experimental import pallas as pl
from jax.experimental.pallas import tpu as pltpu
from jax.experimental.pallas import tpu_sc as plsc

N = 6400000
NUM_SEG = 512
NC, NS, L = 2, 16, 16
NW = NC * NS
B = 12800
NBLK = N // B
STEPS = NBLK // NW
EXTRA = NBLK % NW
SEG_PER_TILE = NUM_SEG // NS


def _make_kernel():
    mesh = plsc.VectorSubcoreMesh(core_axis_name="c", subcore_axis_name="s")

    @functools.partial(
        pl.kernel,
        mesh=mesh,
        compiler_params=pltpu.CompilerParams(needs_layout_passes=False),
        out_type=jax.ShapeDtypeStruct((NC * NUM_SEG,), jnp.float32),
        scratch_types=[
            pltpu.VMEM((2, B), jnp.float32),
            pltpu.VMEM((2, B), jnp.int32),
            pltpu.VMEM((NUM_SEG * L,), jnp.float32),
            pltpu.VMEM((NS * SEG_PER_TILE * L,), jnp.float32),
            pltpu.VMEM((L * L,), jnp.float32),
            pltpu.VMEM((SEG_PER_TILE,), jnp.float32),
            pltpu.VMEM_SHARED((NS * NUM_SEG * L,), jnp.float32),
            pltpu.SemaphoreType.DMA,
            pltpu.SemaphoreType.DMA,
            pltpu.SemaphoreType.DMA,
            pltpu.SemaphoreType.DMA,
        ],
    )
    def seg_sum(x_hbm, b_hbm, out_hbm, xbuf, bbuf, acc2, tmp, mat, res,
                shared, sx0, sb0, sx1, sb1):
        cid = lax.axis_index("c")
        sid = lax.axis_index("s")
        wid = sid * NC + cid
        viota = lax.iota(jnp.int32, L)

        def zbody(i, _):
            acc2[pl.ds(i * L, L)] = jnp.zeros((L,), jnp.float32)
            return _
        lax.fori_loop(0, NUM_SEG * L // L, zbody, 0)

        sems = ((sx0, sb0), (sx1, sb1))

        def dma_start(step, buf):
            off = (wid + NW * step) * B
            pltpu.async_copy(x_hbm.at[pl.ds(off, B)], xbuf.at[buf],
                             sems[buf][0])
            pltpu.async_copy(b_hbm.at[pl.ds(off, B)], bbuf.at[buf],
                             sems[buf][1])

        def dma_wait(buf):
            pltpu.make_async_copy(x_hbm.at[pl.ds(0, B)], xbuf.at[buf],
                                  sems[buf][0]).wait()
            pltpu.make_async_copy(b_hbm.at[pl.ds(0, B)], bbuf.at[buf],
                                  sems[buf][1]).wait()

        UNROLL = 8

        def compute(buf):
            def vbody(i, _):
                for u in range(UNROLL):
                    off = (i * UNROLL + u) * L
                    bv = bbuf[buf, pl.ds(off, L)]
                    xv = xbuf[buf, pl.ds(off, L)]
                    idx = bv * L + viota
                    plsc.addupdate_scatter(acc2, [idx], xv)
                return _
            lax.fori_loop(0, B // L // UNROLL, vbody, 0)

        total_steps = STEPS + 1
        dma_start(0, 0)
        for step in range(total_steps):
            buf = step & 1
            if step == total_steps - 1:
                @pl.when(wid < EXTRA)
                def _():
                    dma_wait(buf)
                    compute(buf)
            else:
                dma_wait(buf)
                nxt = step + 1
                if nxt == total_steps - 1:
                    @pl.when(wid < EXTRA)
                    def _():
                        dma_start(nxt, 1 - buf)
                else:
                    dma_start(nxt, 1 - buf)
                compute(buf)

        PLANE = NUM_SEG * L
        SLICE = SEG_PER_TILE * L
        pltpu.sync_copy(acc2, shared.at[pl.ds(sid * PLANE, PLANE)])
        plsc.subcore_barrier()
        for w in range(NS):
            pltpu.sync_copy(
                shared.at[pl.ds(w * PLANE + sid * SLICE, SLICE)],
                tmp.at[pl.ds(w * SLICE, SLICE)])
        viota16 = viota * L
        for g in range(SEG_PER_TILE // L):
            for j in range(L):
                v = jnp.zeros((L,), jnp.float32)
                for w in range(NS):
                    v = v + tmp[pl.ds(w * SLICE + g * L * L + j * L, L)]
                mat[pl.ds(j * L, L)] = v
            rows = jnp.zeros((L,), jnp.float32)
            for i in range(L):
                rows = rows + plsc.load_gather(mat, [viota16 + i])
            res[pl.ds(g * L, L)] = rows
        pltpu.sync_copy(res, out_hbm.at[pl.ds(cid * NUM_SEG + sid * SEG_PER_TILE,
                                              SEG_PER_TILE)])

    return seg_sum


_seg_sum = _make_kernel()


def kernel(x, batch):
    partials = _seg_sum(x.reshape(N), batch.astype(jnp.int32)).reshape(NC, NUM_SEG)
    return (partials[0] + partials[1]).reshape(NUM_SEG, 1)


# speedup vs baseline: 88.4258x; 1.6615x over previous
import functools

import jax
import jax.numpy as jnp
from jax import lax
from jax.experimental import pallas as pl
from jax.experimental.pallas import tpu as pltpu
from jax.experimental.pallas import tpu_sc as plsc

N = 6400000
NUM_SEG = 512
NC, NS, L = 2, 16, 16
NW = NC * NS
B = 12800
NBLK = N // B
STEPS = NBLK // NW
EXTRA = NBLK % NW
SEG_PER_TILE = NUM_SEG // NS


def _make_kernel():
    mesh = plsc.VectorSubcoreMesh(core_axis_name="c", subcore_axis_name="s")

    @functools.partial(
        pl.kernel,
        mesh=mesh,
        compiler_params=pltpu.CompilerParams(needs_layout_passes=False),
        out_type=jax.ShapeDtypeStruct((NC * NUM_SEG,), jnp.float32),
        scratch_types=[
            pltpu.VMEM((2, B), jnp.float32),
            pltpu.VMEM((2 * B + 128,), jnp.int32),
            pltpu.VMEM((NUM_SEG * L,), jnp.float32),
            pltpu.VMEM((NS * SEG_PER_TILE * L,), jnp.float32),
            pltpu.VMEM((L * L,), jnp.float32),
            pltpu.VMEM((SEG_PER_TILE,), jnp.float32),
            pltpu.VMEM_SHARED((NS * NUM_SEG * L,), jnp.float32),
            pltpu.SemaphoreType.DMA,
            pltpu.SemaphoreType.DMA,
            pltpu.SemaphoreType.DMA,
            pltpu.SemaphoreType.DMA,
        ],
    )
    def seg_sum(x_hbm, b_hbm, out_hbm, xbuf, bbuf, acc2, tmp, mat, res,
                shared, sx0, sb0, sx1, sb1):
        cid = lax.axis_index("c")
        sid = lax.axis_index("s")
        wid = sid * NC + cid
        viota = lax.iota(jnp.int32, L)

        def zbody(i, _):
            acc2[pl.ds(i * L, L)] = jnp.zeros((L,), jnp.float32)
            return _
        lax.fori_loop(0, NUM_SEG * L // L, zbody, 0)

        sems = ((sx0, sb0), (sx1, sb1))

        def dma_start(step, buf):
            off = (wid + NW * step) * B
            pltpu.async_copy(x_hbm.at[pl.ds(off, B)], xbuf.at[buf],
                             sems[buf][0])
            pltpu.async_copy(b_hbm.at[pl.ds(off, B)],
                             bbuf.at[pl.ds(buf * B, B)], sems[buf][1])

        def dma_wait(buf):
            pltpu.make_async_copy(x_hbm.at[pl.ds(0, B)], xbuf.at[buf],
                                  sems[buf][0]).wait()
            pltpu.make_async_copy(b_hbm.at[pl.ds(0, B)],
                                  bbuf.at[pl.ds(buf * B, B)],
                                  sems[buf][1]).wait()

        GB = 256
        NG = B // GB
        is_last = viota == (L - 1)
        not_last = viota < (L - 1)
        zero_v = jnp.zeros((L,), jnp.float32)

        def compute(buf):
            bbase = buf * B
            h0 = bbuf[pl.ds(bbase, L)][0]

            def gbody(g, carry):
                h, vacc = carry
                goff = g * GB
                s0 = bbuf[pl.ds(bbase + goff, L)][0]
                sl = bbuf[pl.ds(bbase + goff + GB - L, L)][L - 1]
                general = s0 != sl
                need_flush = jnp.logical_or(general, s0 != h)

                def do_flush(va):
                    plsc.addupdate_scatter(acc2, [h * L + viota], va)
                    return zero_v
                vacc = lax.cond(need_flush, do_flush, lambda va: va, vacc)
                h = jnp.where(general, sl, s0)

                def fast(va):
                    for u in range(GB // L):
                        va = va + xbuf[buf, pl.ds(goff + u * L, L)]
                    return va

                def slow(va):
                    for u in range(GB // L):
                        off = goff + u * L
                        bv = bbuf[pl.ds(bbase + off, L)]
                        bn = bbuf[pl.ds(bbase + off + 1, L)]
                        xv = xbuf[buf, pl.ds(off, L)]
                        s = plsc.cumsum(xv)
                        m = bv != bn
                        m_end = jnp.logical_or(m, is_last)
                        m_mid = jnp.logical_and(m, not_last)
                        plsc.addupdate_scatter(acc2, [bv * L + viota], s,
                                               mask=m_end)
                        plsc.addupdate_scatter(acc2, [bn * L + viota], -s,
                                               mask=m_mid)
                    return va
                vacc = lax.cond(general, slow, fast, vacc)
                return h, vacc

            h_end, v_end = lax.fori_loop(0, NG, gbody, (h0, zero_v))
            plsc.addupdate_scatter(acc2, [h_end * L + viota], v_end)

        total_steps = STEPS + 1
        dma_start(0, 0)
        for step in range(total_steps):
            buf = step & 1
            if step == total_steps - 1:
                @pl.when(wid < EXTRA)
                def _():
                    dma_wait(buf)
                    compute(buf)
            else:
                dma_wait(buf)
                nxt = step + 1
                if nxt == total_steps - 1:
                    @pl.when(wid < EXTRA)
                    def _():
                        dma_start(nxt, 1 - buf)
                else:
                    dma_start(nxt, 1 - buf)
                compute(buf)

        PLANE = NUM_SEG * L
        SLICE = SEG_PER_TILE * L
        pltpu.sync_copy(acc2, shared.at[pl.ds(sid * PLANE, PLANE)])
        plsc.subcore_barrier()
        for w in range(NS):
            pltpu.sync_copy(
                shared.at[pl.ds(w * PLANE + sid * SLICE, SLICE)],
                tmp.at[pl.ds(w * SLICE, SLICE)])
        viota16 = viota * L
        for g in range(SEG_PER_TILE // L):
            for j in range(L):
                v = jnp.zeros((L,), jnp.float32)
                for w in range(NS):
                    v = v + tmp[pl.ds(w * SLICE + g * L * L + j * L, L)]
                mat[pl.ds(j * L, L)] = v
            rows = jnp.zeros((L,), jnp.float32)
            for i in range(L):
                rows = rows + plsc.load_gather(mat, [viota16 + i])
            res[pl.ds(g * L, L)] = rows
        pltpu.sync_copy(res, out_hbm.at[pl.ds(cid * NUM_SEG + sid * SEG_PER_TILE,
                                              SEG_PER_TILE)])

    return seg_sum


_seg_sum = _make_kernel()


def kernel(x, batch):
    partials = _seg_sum(x.reshape(N), batch.astype(jnp.int32)).reshape(NC, NUM_SEG)
    return (partials[0] + partials[1]).reshape(NUM_SEG, 1)


# speedup vs baseline: 91.8846x; 1.0391x over previous
import functools

import jax
import jax.numpy as jnp
from jax import lax
from jax.experimental import pallas as pl
from jax.experimental.pallas import tpu as pltpu
from jax.experimental.pallas import tpu_sc as plsc

N = 6400000
NUM_SEG = 512
NC, NS, L = 2, 16, 16
NW = NC * NS
B = 12800
NBLK = N // B
STEPS = NBLK // NW
EXTRA = NBLK % NW
SEG_PER_TILE = NUM_SEG // NS


def _make_kernel():
    mesh = plsc.VectorSubcoreMesh(core_axis_name="c", subcore_axis_name="s")

    @functools.partial(
        pl.kernel,
        mesh=mesh,
        compiler_params=pltpu.CompilerParams(needs_layout_passes=False),
        out_type=jax.ShapeDtypeStruct((NC * NUM_SEG,), jnp.float32),
        scratch_types=[
            pltpu.VMEM((2, B), jnp.float32),
            pltpu.VMEM((2 * B + 128,), jnp.int32),
            pltpu.VMEM((NUM_SEG * L,), jnp.float32),
            pltpu.VMEM((NS * SEG_PER_TILE * L,), jnp.float32),
            pltpu.VMEM((L * L,), jnp.float32),
            pltpu.VMEM((SEG_PER_TILE,), jnp.float32),
            pltpu.VMEM_SHARED((NS * NUM_SEG * L,), jnp.float32),
            pltpu.SemaphoreType.DMA,
            pltpu.SemaphoreType.DMA,
            pltpu.SemaphoreType.DMA,
            pltpu.SemaphoreType.DMA,
        ],
    )
    def seg_sum(x_hbm, b_hbm, out_hbm, xbuf, bbuf, acc2, tmp, mat, res,
                shared, sx0, sb0, sx1, sb1):
        cid = lax.axis_index("c")
        sid = lax.axis_index("s")
        wid = sid * NC + cid
        viota = lax.iota(jnp.int32, L)

        def zbody(i, _):
            acc2[pl.ds(i * L, L)] = jnp.zeros((L,), jnp.float32)
            return _
        lax.fori_loop(0, NUM_SEG * L // L, zbody, 0)

        sems = ((sx0, sb0), (sx1, sb1))

        def dma_start(step, buf):
            off = (wid + NW * step) * B
            pltpu.async_copy(x_hbm.at[pl.ds(off, B)], xbuf.at[buf],
                             sems[buf][0])
            pltpu.async_copy(b_hbm.at[pl.ds(off, B)],
                             bbuf.at[pl.ds(buf * B, B)], sems[buf][1])

        def dma_wait(buf):
            pltpu.make_async_copy(x_hbm.at[pl.ds(0, B)], xbuf.at[buf],
                                  sems[buf][0]).wait()
            pltpu.make_async_copy(b_hbm.at[pl.ds(0, B)],
                                  bbuf.at[pl.ds(buf * B, B)],
                                  sems[buf][1]).wait()

        GB = 256
        NG = B // GB
        is_last = viota == (L - 1)
        not_last = viota < (L - 1)
        zero_v = jnp.zeros((L,), jnp.float32)

        def compute(buf):
            bbase = buf * B
            h0_vec = plsc.load_gather(
                bbuf, [jnp.full((L,), bbase, jnp.int32)])

            def gbody(g, carry):
                h_vec, vacc = carry
                gx = g * GB
                goff = bbase + gx
                vb0 = bbuf[pl.ds(goff, L)]
                vbL = bbuf[pl.ds(goff + GB - L, L)]
                general_v = vb0 != vbL
                diff_h = jnp.logical_or(general_v,
                                        jnp.logical_or(vb0 != h_vec,
                                                       vbL != h_vec))
                flush_m = plsc.all_reduce_population_count(diff_h) > 0
                plsc.addupdate_scatter(acc2, [h_vec * L + viota], vacc,
                                       mask=flush_m)
                vacc = jnp.where(flush_m, 0.0, vacc)
                hnew = plsc.load_gather(
                    bbuf, [jnp.full((L,), goff + GB - 1, jnp.int32)])
                h_vec = jnp.where(flush_m, hnew, h_vec)
                ngen = plsc.all_reduce_population_count(general_v)[0]

                def fast(va):
                    vs = [xbuf[buf, pl.ds(gx + u * L, L)]
                          for u in range(GB // L)]
                    while len(vs) > 1:
                        vs = [a + b for a, b in zip(vs[::2], vs[1::2])]
                    return va + vs[0]

                def slow(va):
                    for u in range(GB // L):
                        off = gx + u * L
                        bv = bbuf[pl.ds(bbase + off, L)]
                        bn = bbuf[pl.ds(bbase + off + 1, L)]
                        xv = xbuf[buf, pl.ds(off, L)]
                        s = plsc.cumsum(xv)
                        m = bv != bn
                        m_end = jnp.logical_or(m, is_last)
                        m_mid = jnp.logical_and(m, not_last)
                        plsc.addupdate_scatter(acc2, [bv * L + viota], s,
                                               mask=m_end)
                        plsc.addupdate_scatter(acc2, [bn * L + viota], -s,
                                               mask=m_mid)
                    return va
                vacc = lax.cond(ngen == 0, fast, slow, vacc)
                return h_vec, vacc

            h_end, v_end = lax.fori_loop(0, NG, gbody, (h0_vec, zero_v))
            plsc.addupdate_scatter(acc2, [h_end * L + viota], v_end)

        total_steps = STEPS + 1
        dma_start(0, 0)
        for step in range(total_steps):
            buf = step & 1
            if step == total_steps - 1:
                @pl.when(wid < EXTRA)
                def _():
                    dma_wait(buf)
                    compute(buf)
            else:
                dma_wait(buf)
                nxt = step + 1
                if nxt == total_steps - 1:
                    @pl.when(wid < EXTRA)
                    def _():
                        dma_start(nxt, 1 - buf)
                else:
                    dma_start(nxt, 1 - buf)
                compute(buf)

        PLANE = NUM_SEG * L
        SLICE = SEG_PER_TILE * L
        pltpu.sync_copy(acc2, shared.at[pl.ds(sid * PLANE, PLANE)])
        plsc.subcore_barrier()
        for w in range(NS):
            pltpu.sync_copy(
                shared.at[pl.ds(w * PLANE + sid * SLICE, SLICE)],
                tmp.at[pl.ds(w * SLICE, SLICE)])
        viota16 = viota * L
        for g in range(SEG_PER_TILE // L):
            for j in range(L):
                v = jnp.zeros((L,), jnp.float32)
                for w in range(NS):
                    v = v + tmp[pl.ds(w * SLICE + g * L * L + j * L, L)]
                mat[pl.ds(j * L, L)] = v
            rows = jnp.zeros((L,), jnp.float32)
            for i in range(L):
                rows = rows + plsc.load_gather(mat, [viota16 + i])
            res[pl.ds(g * L, L)] = rows
        pltpu.sync_copy(res, out_hbm.at[pl.ds(cid * NUM_SEG + sid * SEG_PER_TILE,
                                              SEG_PER_TILE)])

    return seg_sum


_seg_sum = _make_kernel()


def kernel(x, batch):
    partials = _seg_sum(x.reshape(N), batch.astype(jnp.int32)).reshape(NC, NUM_SEG)
    return (partials[0] + partials[1]).reshape(NUM_SEG, 1)
